# Initial kernel scaffold; baseline (speedup 1.0000x reference)
import jax
import jax.numpy as jnp
from jax.experimental import pallas as pl


def kernel(x, pos_table):
    raise NotImplementedError("write your pallas kernel here")


# speedup vs baseline: 11.6856x; 11.6856x over previous
import functools

import jax
import jax.numpy as jnp
from jax.experimental import pallas as pl
from jax.experimental.pallas import tpu as pltpu

_BB = 32


def _body(x_ref, tab_ref, o_ref):
    x = x_ref[...]
    rows = tab_ref[1:, :]
    row0 = tab_ref[0:1, :]
    nz = jnp.any(x != 0.0, axis=2)
    pos = jnp.where(nz[:, :, None], rows[None, :, :], row0[None, :, :])
    o_ref[...] = x + pos


def kernel(x, pos_table):
    B, L, D = x.shape
    grid = (B // _BB,)
    return pl.pallas_call(
        _body,
        grid=grid,
        in_specs=[
            pl.BlockSpec((_BB, L, D), lambda i: (i, 0, 0)),
            pl.BlockSpec((L + 1, D), lambda i: (0, 0)),
        ],
        out_specs=pl.BlockSpec((_BB, L, D), lambda i: (i, 0, 0)),
        out_shape=jax.ShapeDtypeStruct((B, L, D), x.dtype),
    )(x, pos_table)
